# baseline (speedup 1.0000x reference)
import functools

import jax
import jax.numpy as jnp
from jax import lax
from jax.experimental import pallas as pl
from jax.experimental.pallas import tpu as pltpu
from jax.experimental.pallas import tpu_sc as plsc

D = 128
NC, NS = 2, 16
NW = NC * NS
CHUNK = 128
NBUF = 2
LANES = 16


def _sc_aggregate(x, adj, n_nodes):
    e = adj.shape[1]
    assert e % CHUNK == 0
    n_chunks = e // CHUNK
    U = 4
    base_rounds = n_chunks // NW // U
    rem = n_chunks - base_rounds * U * NW
    assert rem % U == 0, "edge count must split into U-chunk rounds"
    extra_tiles = rem // U
    assert extra_tiles <= NW and base_rounds >= 1
    n_acc = ((n_nodes + NS * CHUNK - 1) // (NS * CHUNK)) * (NS * CHUNK)
    rows_per_tile = n_acc // NS
    n_pieces = rows_per_tile // CHUNK
    mesh = plsc.VectorSubcoreMesh(core_axis_name="c", subcore_axis_name="s")

    @functools.partial(
        pl.kernel,
        out_type=(
            jax.ShapeDtypeStruct((NC, n_acc, D), jnp.float32),
            jax.ShapeDtypeStruct((NW, n_acc), jnp.float32),
        ),
        mesh=mesh,
        scratch_types=(
            [pltpu.VMEM_SHARED((n_acc, D), jnp.float32)]
            + [pltpu.VMEM((n_acc,), jnp.float32)]
            + [pltpu.VMEM((2, CHUNK), jnp.int32) for _ in range(2 * NBUF)]
            + [pltpu.VMEM((CHUNK, D), jnp.float32) for _ in range(NBUF)]
            + [pltpu.SemaphoreType.DMA for _ in range(2 * NBUF)]
        ),
        compiler_params=pltpu.CompilerParams(needs_layout_passes=False),
    )
    def agg(x_hbm, adj_hbm, out_hbm, deg_hbm, acc, hist, *bufs):
        eidxs = bufs[:2 * NBUF]
        rows = bufs[2 * NBUF:3 * NBUF]
        gsems = bufs[3 * NBUF:4 * NBUF]
        ssems = bufs[4 * NBUF:5 * NBUF]
        c = lax.axis_index("c")
        s = lax.axis_index("s")

        zvf = jnp.zeros((LANES,), jnp.float32)

        def fill(i, carry):
            for j in range(D // LANES):
                rows[0][i, pl.ds(j * LANES, LANES)] = zvf
            return carry

        lax.fori_loop(0, CHUNK, fill, 0)
        r0 = s * rows_per_tile
        for k in range(n_pieces):
            pltpu.sync_copy(rows[0], acc.at[pl.ds(r0 + k * CHUNK, CHUNK)])

        wid = c * NS + s
        chunk0 = wid * base_rounds * U + U * jnp.minimum(wid, extra_tiles)
        my_chunks = U * (base_rounds + jnp.where(wid < extra_tiles, 1, 0))
        ones16 = jnp.ones((LANES,), jnp.float32)

        for q in range(NBUF):
            pltpu.sync_copy(adj_hbm.at[:, pl.ds((chunk0 + q) * CHUNK, CHUNK)],
                            eidxs[q])
            pltpu.async_copy(x_hbm.at[eidxs[q].at[0]], rows[q], gsems[q])

        def fill_hist(i, carry):
            hist[pl.ds(i * LANES, LANES)] = zvf
            return carry

        lax.fori_loop(0, n_acc // LANES, fill_hist, 0)
        plsc.subcore_barrier()

        def outer(t, carry):
            for b in range(U):
                g = t * U + b
                r = b % NBUF
                eidx, row = eidxs[b % (2 * NBUF)], rows[r]
                pltpu.make_async_copy(x_hbm.at[eidx.at[0]], row, gsems[r]).wait()
                scat = pltpu.async_copy(row, acc.at[eidx.at[1]], ssems[r],
                                        add=True)
                for j in range(CHUNK // LANES):
                    idx16 = eidx[1, pl.ds(j * LANES, LANES)]
                    plsc.addupdate_scatter(hist, [idx16], ones16)

                @pl.when(g + NBUF < my_chunks)
                def _():
                    nq = (b + NBUF) % (2 * NBUF)
                    cb = (chunk0 + g + NBUF) * CHUNK
                    pltpu.sync_copy(adj_hbm.at[:, pl.ds(cb, CHUNK)], eidxs[nq])
                    scat.wait()
                    pltpu.async_copy(x_hbm.at[eidxs[nq].at[0]], row, gsems[r])

                @pl.when(g + NBUF >= my_chunks)
                def _():
                    scat.wait()

            return carry

        lax.fori_loop(0, base_rounds + jnp.where(wid < extra_tiles, 1, 0),
                      outer, 0)
        plsc.subcore_barrier()

        pltpu.sync_copy(acc.at[pl.ds(r0, rows_per_tile)],
                        out_hbm.at[c, pl.ds(r0, rows_per_tile)])
        pltpu.sync_copy(hist, deg_hbm.at[wid])

    return agg(x, adj)


def _tc_head(x, partial, degs, W, b):
    n = x.shape[0]
    R = 1024
    grid = ((n + R - 1) // R,)

    def body(x_ref, p_ref, d_ref, w_ref, b_ref, o_ref):
        xb = x_ref[...]
        p = p_ref[...]
        ssum = p[0] + p[1]
        deg = jnp.sum(d_ref[...], axis=0)[:, None]
        mean = ssum / jnp.maximum(deg, 1.0)
        w = w_ref[...]
        h = (
            jnp.dot(xb, w[:D], preferred_element_type=jnp.float32)
            + jnp.dot(mean, w[D:], preferred_element_type=jnp.float32)
            + b_ref[...]
        )
        h = jnp.maximum(h, 0.0)
        nrm = jnp.sqrt(jnp.sum(h * h, axis=1, keepdims=True))
        o_ref[...] = h / jnp.maximum(nrm, 1e-12)

    return pl.pallas_call(
        body,
        grid=grid,
        in_specs=[
            pl.BlockSpec((R, D), lambda i: (i, 0)),
            pl.BlockSpec((NC, R, D), lambda i: (0, i, 0)),
            pl.BlockSpec((NW, R), lambda i: (0, i)),
            pl.BlockSpec((2 * D, D), lambda i: (0, 0)),
            pl.BlockSpec((1, D), lambda i: (0, 0)),
        ],
        out_specs=pl.BlockSpec((R, D), lambda i: (i, 0)),
        out_shape=jax.ShapeDtypeStruct((n, D), jnp.float32),
    )(x, partial, degs, W, b.reshape(1, D))


def kernel(input_matrix, adjacency_coo_matrix, W, b):
    x = input_matrix
    n = x.shape[0]
    adj = adjacency_coo_matrix.astype(jnp.int32)
    partial, degs = _sc_aggregate(x, adj, n)
    return _tc_head(x, partial, degs, W, b)


# speedup vs baseline: 1.0050x; 1.0050x over previous
import functools

import jax
import jax.numpy as jnp
from jax import lax
from jax.experimental import pallas as pl
from jax.experimental.pallas import tpu as pltpu
from jax.experimental.pallas import tpu_sc as plsc

D = 128
NC, NS = 2, 16
NW = NC * NS
CHUNK = 128
NBUF = 2
LANES = 16


def _sc_aggregate(x, adj, n_nodes):
    e = adj.shape[1]
    assert e % CHUNK == 0
    n_chunks = e // CHUNK
    U = 4
    base_rounds = n_chunks // NW // U
    rem = n_chunks - base_rounds * U * NW
    assert rem % U == 0, "edge count must split into U-chunk rounds"
    extra_tiles = rem // U
    assert extra_tiles <= NW and base_rounds >= 1
    n_acc = ((n_nodes + NS * CHUNK - 1) // (NS * CHUNK)) * (NS * CHUNK)
    rows_per_tile = n_acc // NS
    n_pieces = rows_per_tile // CHUNK
    mesh = plsc.VectorSubcoreMesh(core_axis_name="c", subcore_axis_name="s")

    @functools.partial(
        pl.kernel,
        out_type=(
            jax.ShapeDtypeStruct((NC, n_acc, D), jnp.float32),
            jax.ShapeDtypeStruct((NW, n_acc), jnp.float32),
        ),
        mesh=mesh,
        scratch_types=(
            [pltpu.VMEM_SHARED((n_acc, D), jnp.bfloat16)]
            + [pltpu.VMEM((n_acc,), jnp.float32)]
            + [pltpu.VMEM((2, CHUNK), jnp.int32) for _ in range(2 * NBUF)]
            + [pltpu.VMEM((CHUNK, D), jnp.bfloat16) for _ in range(NBUF)]
            + [pltpu.VMEM((CHUNK, D), jnp.float32)]
            + [pltpu.SemaphoreType.DMA for _ in range(2 * NBUF)]
        ),
        compiler_params=pltpu.CompilerParams(
            needs_layout_passes=False, use_tc_tiling_on_sc=False),
    )
    def agg(x_hbm, adj_hbm, out_hbm, deg_hbm, acc, hist, *bufs):
        eidxs = bufs[:2 * NBUF]
        rows = bufs[2 * NBUF:3 * NBUF]
        frow = bufs[3 * NBUF]
        gsems = bufs[3 * NBUF + 1:4 * NBUF + 1]
        ssems = bufs[4 * NBUF + 1:5 * NBUF + 1]
        c = lax.axis_index("c")
        s = lax.axis_index("s")

        zvf = jnp.zeros((LANES,), jnp.float32)

        zvb = jnp.zeros((2 * LANES,), jnp.bfloat16)

        def fill(i, carry):
            for j in range(D // (2 * LANES)):
                rows[0][i, pl.ds(j * 2 * LANES, 2 * LANES)] = zvb
            return carry

        lax.fori_loop(0, CHUNK, fill, 0)
        r0 = s * rows_per_tile
        for k in range(n_pieces):
            pltpu.sync_copy(rows[0], acc.at[pl.ds(r0 + k * CHUNK, CHUNK)])

        wid = c * NS + s
        chunk0 = wid * base_rounds * U + U * jnp.minimum(wid, extra_tiles)
        my_chunks = U * (base_rounds + jnp.where(wid < extra_tiles, 1, 0))
        ones16 = jnp.ones((LANES,), jnp.float32)

        for q in range(NBUF):
            pltpu.sync_copy(adj_hbm.at[:, pl.ds((chunk0 + q) * CHUNK, CHUNK)],
                            eidxs[q])
            pltpu.async_copy(x_hbm.at[eidxs[q].at[0]], rows[q], gsems[q])

        def fill_hist(i, carry):
            hist[pl.ds(i * LANES, LANES)] = zvf
            return carry

        lax.fori_loop(0, n_acc // LANES, fill_hist, 0)
        plsc.subcore_barrier()

        def outer(t, carry):
            for b in range(U):
                g = t * U + b
                r = b % NBUF
                eidx, row = eidxs[b % (2 * NBUF)], rows[r]
                pltpu.make_async_copy(x_hbm.at[eidx.at[0]], row, gsems[r]).wait()
                scat = pltpu.async_copy(row, acc.at[eidx.at[1]], ssems[r],
                                        add=True)
                for j in range(CHUNK // LANES):
                    idx16 = eidx[1, pl.ds(j * LANES, LANES)]
                    plsc.addupdate_scatter(hist, [idx16], ones16)

                @pl.when(g + NBUF < my_chunks)
                def _():
                    nq = (b + NBUF) % (2 * NBUF)
                    cb = (chunk0 + g + NBUF) * CHUNK
                    pltpu.sync_copy(adj_hbm.at[:, pl.ds(cb, CHUNK)], eidxs[nq])
                    scat.wait()
                    pltpu.async_copy(x_hbm.at[eidxs[nq].at[0]], row, gsems[r])

                @pl.when(g + NBUF >= my_chunks)
                def _():
                    scat.wait()

            return carry

        lax.fori_loop(0, base_rounds + jnp.where(wid < extra_tiles, 1, 0),
                      outer, 0)
        plsc.subcore_barrier()

        col_even = lax.iota(jnp.int32, LANES) * 2
        col_odd = col_even + 1
        lo_mask = jnp.full((LANES,), -65536, jnp.int32)

        def cvt_row(i, carry):
            irow = jnp.full((LANES,), i, jnp.int32)
            for j in range(D // (2 * LANES)):
                wbits = plsc.bitcast(rows[0][i, pl.ds(j * 2 * LANES, 2 * LANES)],
                                     jnp.int32)
                f_even = plsc.bitcast(wbits << 16, jnp.float32)
                f_odd = plsc.bitcast(wbits & lo_mask, jnp.float32)
                base = jnp.full((LANES,), j * 2 * LANES, jnp.int32)
                plsc.store_scatter(frow, [irow, base + col_even], f_even)
                plsc.store_scatter(frow, [irow, base + col_odd], f_odd)
            return carry

        for k in range(n_pieces):
            pltpu.sync_copy(acc.at[pl.ds(r0 + k * CHUNK, CHUNK)], rows[0])
            lax.fori_loop(0, CHUNK, cvt_row, 0)
            pltpu.sync_copy(frow, out_hbm.at[c, pl.ds(r0 + k * CHUNK, CHUNK)])
        pltpu.sync_copy(hist, deg_hbm.at[wid])

    return agg(x, adj)


def _tc_head(x, partial, degs, W, b):
    n = x.shape[0]
    R = 1024
    grid = ((n + R - 1) // R,)

    def body(x_ref, p_ref, d_ref, w_ref, b_ref, o_ref):
        xb = x_ref[...]
        p = p_ref[...]
        ssum = p[0] + p[1]
        deg = jnp.sum(d_ref[...], axis=0)[:, None]
        mean = ssum / jnp.maximum(deg, 1.0)
        w = w_ref[...]
        h = (
            jnp.dot(xb, w[:D], preferred_element_type=jnp.float32)
            + jnp.dot(mean, w[D:], preferred_element_type=jnp.float32)
            + b_ref[...]
        )
        h = jnp.maximum(h, 0.0)
        nrm = jnp.sqrt(jnp.sum(h * h, axis=1, keepdims=True))
        o_ref[...] = h / jnp.maximum(nrm, 1e-12)

    return pl.pallas_call(
        body,
        grid=grid,
        in_specs=[
            pl.BlockSpec((R, D), lambda i: (i, 0)),
            pl.BlockSpec((NC, R, D), lambda i: (0, i, 0)),
            pl.BlockSpec((NW, R), lambda i: (0, i)),
            pl.BlockSpec((2 * D, D), lambda i: (0, 0)),
            pl.BlockSpec((1, D), lambda i: (0, 0)),
        ],
        out_specs=pl.BlockSpec((R, D), lambda i: (i, 0)),
        out_shape=jax.ShapeDtypeStruct((n, D), jnp.float32),
    )(x, partial, degs, W, b.reshape(1, D))


def kernel(input_matrix, adjacency_coo_matrix, W, b):
    x = input_matrix
    n = x.shape[0]
    adj = adjacency_coo_matrix.astype(jnp.int32)
    partial, degs = _sc_aggregate(x.astype(jnp.bfloat16), adj, n)
    return _tc_head(x, partial, degs, W, b)


# speedup vs baseline: 1.0069x; 1.0019x over previous
import functools

import jax
import jax.numpy as jnp
from jax import lax
from jax.experimental import pallas as pl
from jax.experimental.pallas import tpu as pltpu
from jax.experimental.pallas import tpu_sc as plsc

D = 128
NC, NS = 2, 16
NW = NC * NS
CHUNK = 128
RING = 4
LEAD = 2
LANES = 16


def _sc_aggregate(x, adj, n_nodes):
    e = adj.shape[1]
    assert e % CHUNK == 0
    n_chunks = e // CHUNK
    U = RING
    base_rounds = n_chunks // NW // U
    rem = n_chunks - base_rounds * U * NW
    assert rem % U == 0, "edge count must split into U-chunk rounds"
    extra_tiles = rem // U
    assert extra_tiles <= NW and base_rounds >= 1
    n_acc = ((n_nodes + NS * CHUNK - 1) // (NS * CHUNK)) * (NS * CHUNK)
    rows_per_tile = n_acc // NS
    n_pieces = rows_per_tile // CHUNK
    mesh = plsc.VectorSubcoreMesh(core_axis_name="c", subcore_axis_name="s")

    @functools.partial(
        pl.kernel,
        out_type=(
            jax.ShapeDtypeStruct((NC, n_acc, D), jnp.float32),
            jax.ShapeDtypeStruct((NW, n_acc), jnp.float32),
        ),
        mesh=mesh,
        scratch_types=(
            [pltpu.VMEM_SHARED((n_acc, D), jnp.bfloat16)]
            + [pltpu.VMEM((n_acc,), jnp.float32)]
            + [pltpu.VMEM((2, CHUNK), jnp.int32) for _ in range(RING)]
            + [pltpu.VMEM((CHUNK, D), jnp.bfloat16) for _ in range(RING)]
            + [pltpu.VMEM((CHUNK, D), jnp.float32)]
            + [pltpu.SemaphoreType.DMA for _ in range(2 * RING)]
        ),
        compiler_params=pltpu.CompilerParams(
            needs_layout_passes=False, use_tc_tiling_on_sc=False),
    )
    def agg(x_hbm, adj_hbm, out_hbm, deg_hbm, acc, hist, *bufs):
        eidxs = bufs[:RING]
        rows = bufs[RING:2 * RING]
        frow = bufs[2 * RING]
        gsems = bufs[2 * RING + 1:3 * RING + 1]
        ssems = bufs[3 * RING + 1:4 * RING + 1]
        c = lax.axis_index("c")
        s = lax.axis_index("s")

        zvf = jnp.zeros((LANES,), jnp.float32)

        zvb = jnp.zeros((2 * LANES,), jnp.bfloat16)

        def fill(i, carry):
            for j in range(D // (2 * LANES)):
                rows[0][i, pl.ds(j * 2 * LANES, 2 * LANES)] = zvb
            return carry

        lax.fori_loop(0, CHUNK, fill, 0)
        r0 = s * rows_per_tile
        for k in range(n_pieces):
            pltpu.sync_copy(rows[0], acc.at[pl.ds(r0 + k * CHUNK, CHUNK)])

        wid = c * NS + s
        chunk0 = wid * base_rounds * U + U * jnp.minimum(wid, extra_tiles)
        my_chunks = U * (base_rounds + jnp.where(wid < extra_tiles, 1, 0))
        ones16 = jnp.ones((LANES,), jnp.float32)

        for q in range(LEAD):
            pltpu.sync_copy(adj_hbm.at[:, pl.ds((chunk0 + q) * CHUNK, CHUNK)],
                            eidxs[q])
            pltpu.async_copy(x_hbm.at[eidxs[q].at[0]], rows[q], gsems[q])

        def fill_hist(i, carry):
            hist[pl.ds(i * LANES, LANES)] = zvf
            return carry

        lax.fori_loop(0, n_acc // LANES, fill_hist, 0)
        plsc.subcore_barrier()

        def outer(t, carry):
            for b in range(U):
                g = t * U + b
                eidx, row = eidxs[b], rows[b]
                pltpu.make_async_copy(x_hbm.at[eidx.at[0]], row, gsems[b]).wait()
                pltpu.async_copy(row, acc.at[eidx.at[1]], ssems[b], add=True)
                for j in range(CHUNK // LANES):
                    idx16 = eidx[1, pl.ds(j * LANES, LANES)]
                    plsc.addupdate_scatter(hist, [idx16], ones16)

                nb = (b + LEAD) % RING
                neidx, nrow = eidxs[nb], rows[nb]

                @pl.when(g + LEAD < my_chunks)
                def _():
                    @pl.when(g >= RING - LEAD)
                    def _():
                        pltpu.make_async_copy(
                            nrow, acc.at[neidx.at[1]], ssems[nb]).wait()

                    cb = (chunk0 + g + LEAD) * CHUNK
                    pltpu.sync_copy(adj_hbm.at[:, pl.ds(cb, CHUNK)], neidx)
                    pltpu.async_copy(x_hbm.at[neidx.at[0]], nrow, gsems[nb])

            return carry

        lax.fori_loop(0, base_rounds + jnp.where(wid < extra_tiles, 1, 0),
                      outer, 0)
        for b in range(RING):
            pltpu.make_async_copy(rows[b], acc.at[eidxs[b].at[1]],
                                  ssems[b]).wait()
        plsc.subcore_barrier()

        col_even = lax.iota(jnp.int32, LANES) * 2
        col_odd = col_even + 1
        lo_mask = jnp.full((LANES,), -65536, jnp.int32)

        def cvt_row(i, carry):
            irow = jnp.full((LANES,), i, jnp.int32)
            for j in range(D // (2 * LANES)):
                wbits = plsc.bitcast(rows[0][i, pl.ds(j * 2 * LANES, 2 * LANES)],
                                     jnp.int32)
                f_even = plsc.bitcast(wbits << 16, jnp.float32)
                f_odd = plsc.bitcast(wbits & lo_mask, jnp.float32)
                base = jnp.full((LANES,), j * 2 * LANES, jnp.int32)
                plsc.store_scatter(frow, [irow, base + col_even], f_even)
                plsc.store_scatter(frow, [irow, base + col_odd], f_odd)
            return carry

        for k in range(n_pieces):
            pltpu.sync_copy(acc.at[pl.ds(r0 + k * CHUNK, CHUNK)], rows[0])
            lax.fori_loop(0, CHUNK, cvt_row, 0)
            pltpu.sync_copy(frow, out_hbm.at[c, pl.ds(r0 + k * CHUNK, CHUNK)])
        pltpu.sync_copy(hist, deg_hbm.at[wid])

    return agg(x, adj)


def _tc_head(x, partial, degs, W, b):
    n = x.shape[0]
    R = 1024
    grid = ((n + R - 1) // R,)

    def body(x_ref, p_ref, d_ref, w_ref, b_ref, o_ref):
        xb = x_ref[...]
        p = p_ref[...]
        ssum = p[0] + p[1]
        deg = jnp.sum(d_ref[...], axis=0)[:, None]
        mean = ssum / jnp.maximum(deg, 1.0)
        w = w_ref[...]
        h = (
            jnp.dot(xb, w[:D], preferred_element_type=jnp.float32)
            + jnp.dot(mean, w[D:], preferred_element_type=jnp.float32)
            + b_ref[...]
        )
        h = jnp.maximum(h, 0.0)
        nrm = jnp.sqrt(jnp.sum(h * h, axis=1, keepdims=True))
        o_ref[...] = h / jnp.maximum(nrm, 1e-12)

    return pl.pallas_call(
        body,
        grid=grid,
        in_specs=[
            pl.BlockSpec((R, D), lambda i: (i, 0)),
            pl.BlockSpec((NC, R, D), lambda i: (0, i, 0)),
            pl.BlockSpec((NW, R), lambda i: (0, i)),
            pl.BlockSpec((2 * D, D), lambda i: (0, 0)),
            pl.BlockSpec((1, D), lambda i: (0, 0)),
        ],
        out_specs=pl.BlockSpec((R, D), lambda i: (i, 0)),
        out_shape=jax.ShapeDtypeStruct((n, D), jnp.float32),
    )(x, partial, degs, W, b.reshape(1, D))


def kernel(input_matrix, adjacency_coo_matrix, W, b):
    x = input_matrix
    n = x.shape[0]
    adj = adjacency_coo_matrix.astype(jnp.int32)
    partial, degs = _sc_aggregate(x.astype(jnp.bfloat16), adj, n)
    return _tc_head(x, partial, degs, W, b)
